# baseline (speedup 1.0000x reference)
import functools

import jax
import jax.numpy as jnp
from jax import lax
from jax.experimental import pallas as pl
from jax.experimental.pallas import tpu as pltpu
from jax.experimental.pallas import tpu_sc as plsc

N = 50000
E = 800000
G = 64
H = 64
HH = H // 2

NC = 2
NS = 16
K = 128

CHUNKS = 396
EPT = CHUNKS * K
E_PAD = EPT * NS
BC = 6
NBLK = CHUNKS // BC
ZR = 56
N_PAD = 50176
NPT = N_PAD // NS
WPT = 3136

R = 10000
NB = N // R
SR = 2000
NSR = N // SR

_mesh = plsc.VectorSubcoreMesh(core_axis_name="c", subcore_axis_name="s")


@functools.partial(
    pl.kernel,
    out_type=jax.ShapeDtypeStruct((2 * N, HH), jnp.float32),
    mesh=_mesh,
    scratch_types=[
        pltpu.VMEM_SHARED((N_PAD, HH), jnp.float32),
        pltpu.VMEM((ZR, HH), jnp.float32),
        pltpu.VMEM((BC, K), jnp.int32),
        pltpu.VMEM((BC, K), jnp.int32),
        pltpu.VMEM((BC, K), jnp.int32),
        pltpu.VMEM((BC, K), jnp.int32),
        pltpu.VMEM((BC, K, HH), jnp.float32),
        pltpu.SemaphoreType.DMA((BC,)),
        pltpu.SemaphoreType.DMA((BC,)),
    ],
    compiler_params=pltpu.CompilerParams(use_tc_tiling_on_sc=False),
)
def _agg_sc(h2, idx2, dstp, out, acc, zbuf, isA, idA, isB, idB, rows, sem_g,
            sem_s):
    c = lax.axis_index("c")
    s = lax.axis_index("s")

    ib = c * (E_PAD // K) + s * CHUNKS
    db = s * CHUNKS

    def load_idx(blkno, i_buf, d_buf):
        pltpu.sync_copy(idx2.at[pl.ds(pl.multiple_of(ib + blkno * BC, 2),
                                      BC)], i_buf)
        pltpu.sync_copy(dstp.at[pl.ds(pl.multiple_of(db + blkno * BC, 2),
                                      BC)], d_buf)

    load_idx(0, isA, idA)
    for j in range(BC):
        pltpu.async_copy(h2.at[isA.at[j]], rows.at[j], sem_g.at[j])

    zero = jnp.zeros((16,), jnp.float32)

    def zb(i, carry):
        zbuf[i, 0:16] = zero
        zbuf[i, 16:32] = zero
        return carry

    lax.fori_loop(0, ZR, zb, 0)

    def za(k, carry):
        zoff = pl.multiple_of(s * NPT + k * ZR, 8)
        pltpu.sync_copy(zbuf, acc.at[pl.ds(zoff, ZR)])
        return carry

    lax.fori_loop(0, NPT // ZR, za, 0)
    plsc.subcore_barrier()

    def step(t, carry):
        load_idx(2 * t + 1, isB, idB)
        for j in range(BC):
            pltpu.make_async_copy(h2.at[isA.at[j]], rows.at[j],
                                  sem_g.at[j]).wait()
            pltpu.async_copy(rows.at[j], acc.at[idA.at[j]], sem_s.at[j],
                             add=True)
        for j in range(BC):
            pltpu.make_async_copy(rows.at[j], acc.at[idA.at[j]],
                                  sem_s.at[j]).wait()
            pltpu.async_copy(h2.at[isB.at[j]], rows.at[j], sem_g.at[j])

        @pl.when(t < NBLK // 2 - 1)
        def _():
            load_idx(2 * t + 2, isA, idA)
            for j in range(BC):
                pltpu.make_async_copy(h2.at[isB.at[j]], rows.at[j],
                                      sem_g.at[j]).wait()
                pltpu.async_copy(rows.at[j], acc.at[idB.at[j]], sem_s.at[j],
                                 add=True)
            for j in range(BC):
                pltpu.make_async_copy(rows.at[j], acc.at[idB.at[j]],
                                      sem_s.at[j]).wait()
                pltpu.async_copy(h2.at[isA.at[j]], rows.at[j], sem_g.at[j])

        @pl.when(t == NBLK // 2 - 1)
        def _():
            for j in range(BC):
                pltpu.make_async_copy(h2.at[isB.at[j]], rows.at[j],
                                      sem_g.at[j]).wait()
                pltpu.async_copy(rows.at[j], acc.at[idB.at[j]], sem_s.at[j],
                                 add=True)
            for j in range(BC):
                pltpu.make_async_copy(rows.at[j], acc.at[idB.at[j]],
                                      sem_s.at[j]).wait()

        return carry

    lax.fori_loop(0, NBLK // 2, step, 0)
    plsc.subcore_barrier()

    wbase = pl.multiple_of(jnp.minimum(s * WPT, N - WPT), 8)
    pltpu.sync_copy(acc.at[pl.ds(wbase, WPT)],
                    out.at[pl.ds(pl.multiple_of(c * N + wbase, 8), WPT)])


def _sc_agg(h, idx2, dstp):
    return _agg_sc(h.reshape(2 * N, HH), idx2, dstp)


def _mm_in(xp, Wp):
    def body(x_ref, w_ref, o_ref):
        o_ref[...] = jnp.dot(x_ref[...], w_ref[...],
                             preferred_element_type=jnp.float32)

    return pl.pallas_call(
        body,
        grid=(NB,),
        in_specs=[pl.BlockSpec((R, 8), lambda i: (i, 0)),
                  pl.BlockSpec((8, H), lambda i: (0, 0))],
        out_specs=pl.BlockSpec((R, H), lambda i: (i, 0)),
        out_shape=jax.ShapeDtypeStruct((N, H), jnp.float32),
    )(xp, Wp)


def _addrelu(y, g2):
    def body(y_ref, g0_ref, g1_ref, o_ref):
        g = jnp.concatenate([g0_ref[...], g1_ref[...]], axis=1)
        o_ref[...] = jnp.maximum(y_ref[...] + g, 0.0)

    return pl.pallas_call(
        body,
        grid=(NB,),
        in_specs=[pl.BlockSpec((R, H), lambda i: (i, 0)),
                  pl.BlockSpec((R, HH), lambda i: (i, 0)),
                  pl.BlockSpec((R, HH), lambda i: (i + NB, 0))],
        out_specs=pl.BlockSpec((R, H), lambda i: (i, 0)),
        out_shape=jax.ShapeDtypeStruct((N, H), jnp.float32),
    )(y, g2, g2)


def _layer(h, g2, W):
    def body(h_ref, g0_ref, g1_ref, w_ref, o_ref):
        t0 = h_ref[:, 0:HH] + g0_ref[...]
        t1 = h_ref[:, HH:H] + g1_ref[...]
        acc = jnp.dot(t0, w_ref[0:HH, :], preferred_element_type=jnp.float32)
        acc += jnp.dot(t1, w_ref[HH:H, :], preferred_element_type=jnp.float32)
        o_ref[...] = jnp.maximum(acc, 0.0)

    return pl.pallas_call(
        body,
        grid=(NB,),
        in_specs=[pl.BlockSpec((R, H), lambda i: (i, 0)),
                  pl.BlockSpec((R, HH), lambda i: (i, 0)),
                  pl.BlockSpec((R, HH), lambda i: (i + NB, 0)),
                  pl.BlockSpec((H, H), lambda i: (0, 0))],
        out_specs=pl.BlockSpec((R, H), lambda i: (i, 0)),
        out_shape=jax.ShapeDtypeStruct((N, H), jnp.float32),
    )(h, g2, g2, W)


def _branch(e, g2, Wp0, Wv0):
    def body(h_ref, g0_ref, g1_ref, wp_ref, wv_ref, op_ref, ov_ref):
        t0 = h_ref[:, 0:HH] + g0_ref[...]
        t1 = h_ref[:, HH:H] + g1_ref[...]
        ap = jnp.dot(t0, wp_ref[0:HH, :], preferred_element_type=jnp.float32)
        ap += jnp.dot(t1, wp_ref[HH:H, :], preferred_element_type=jnp.float32)
        op_ref[...] = jnp.maximum(ap, 0.0)
        av = jnp.dot(t0, wv_ref[0:HH, :], preferred_element_type=jnp.float32)
        av += jnp.dot(t1, wv_ref[HH:H, :], preferred_element_type=jnp.float32)
        ov_ref[...] = jnp.maximum(av, 0.0)

    return pl.pallas_call(
        body,
        grid=(NB,),
        in_specs=[pl.BlockSpec((R, H), lambda i: (i, 0)),
                  pl.BlockSpec((R, HH), lambda i: (i, 0)),
                  pl.BlockSpec((R, HH), lambda i: (i + NB, 0)),
                  pl.BlockSpec((H, H), lambda i: (0, 0)),
                  pl.BlockSpec((H, H), lambda i: (0, 0))],
        out_specs=[pl.BlockSpec((R, H), lambda i: (i, 0)),
                   pl.BlockSpec((R, H), lambda i: (i, 0))],
        out_shape=[jax.ShapeDtypeStruct((N, H), jnp.float32),
                   jax.ShapeDtypeStruct((N, H), jnp.float32)],
    )(e, g2, g2, Wp0, Wv0)


def _seg_all(pi, vemb, bcol, Wl, bl2):
    def body(pi_ref, v_ref, b_ref, wl_ref, bl_ref, lp_ref, val_ref,
             m_acc, s_acc, d_acc):
        i = pl.program_id(0)
        giota = lax.broadcasted_iota(jnp.int32, (G, 1), 0)
        giota_row = lax.broadcasted_iota(jnp.int32, (1, G), 1)
        b = b_ref[...]
        onehot = (b == giota_row).astype(jnp.float32)

        @pl.when(i == 0)
        def _():
            m_acc[...] = jnp.full((G, H), -3e38, jnp.float32)
            s_acc[...] = jnp.zeros((G, H), jnp.float32)
            d_acc[...] = jnp.zeros((G, H), jnp.float32)

        @pl.when(i < NSR)
        def _():
            s_acc[...] += lax.dot_general(
                onehot, v_ref[...], (((0,), (0,)), ((), ())),
                preferred_element_type=jnp.float32)
            pi_blk = pi_ref[...]
            g_lo = b_ref[0, 0]
            g_hi = b_ref[SR - 1, 0]

            def gbody(g, carry):
                mask = b == g
                vals = jnp.where(mask, pi_blk, -3e38)
                row = jnp.max(vals, axis=0, keepdims=True)
                sel = giota == g
                m_acc[...] = jnp.where(
                    sel,
                    jnp.maximum(m_acc[...], jnp.broadcast_to(row, (G, H))),
                    m_acc[...])
                return carry

            lax.fori_loop(g_lo, g_hi + 1, gbody, 0)

        @pl.when((i >= NSR) & (i < 2 * NSR))
        def _():
            msel = jnp.dot(onehot, m_acc[...],
                           preferred_element_type=jnp.float32)
            e = jnp.exp(pi_ref[...] - msel)
            d_acc[...] += lax.dot_general(
                onehot, e, (((0,), (0,)), ((), ())),
                preferred_element_type=jnp.float32)

        @pl.when(i >= 2 * NSR)
        def _():
            shift = m_acc[...] + jnp.log(jnp.maximum(d_acc[...], 1e-30))
            ssel = jnp.dot(onehot, shift, preferred_element_type=jnp.float32)
            lp_ref[...] = pi_ref[...] - ssel

        @pl.when(i == 3 * NSR - 1)
        def _():
            v = jnp.dot(s_acc[...], wl_ref[...],
                        preferred_element_type=jnp.float32) + bl_ref[0, 0]
            val_ref[...] = 1.0 / (1.0 + jnp.exp(-v))

    chunk = lambda i: (lax.rem(i, NSR), 0)
    return pl.pallas_call(
        body,
        grid=(3 * NSR,),
        in_specs=[pl.BlockSpec((SR, H), chunk),
                  pl.BlockSpec((SR, H), chunk),
                  pl.BlockSpec((SR, 1), chunk),
                  pl.BlockSpec((H, 1), lambda i: (0, 0)),
                  pl.BlockSpec((1, 1), lambda i: (0, 0))],
        out_specs=[pl.BlockSpec((SR, H),
                                lambda i: (jnp.where(i >= 2 * NSR,
                                                     i - 2 * NSR, 0), 0)),
                   pl.BlockSpec((G, 1), lambda i: (0, 0))],
        out_shape=[jax.ShapeDtypeStruct((N, H), jnp.float32),
                   jax.ShapeDtypeStruct((G, 1), jnp.float32)],
        scratch_shapes=[pltpu.VMEM((G, H), jnp.float32),
                        pltpu.VMEM((G, H), jnp.float32),
                        pltpu.VMEM((G, H), jnp.float32)],
    )(pi, vemb, bcol, Wl, bl2)


def kernel(x, edge_index, batch, Wb0, Wb1, Wb2, Wp0, Wp1, Wv0, Wv1, Wl, bl):
    src = edge_index[0]
    dst = edge_index[1]
    pad = E_PAD - E
    src_p = jnp.concatenate([src, jnp.zeros((pad,), jnp.int32)])
    dst_p = jnp.concatenate([dst, jnp.full((pad,), N, jnp.int32)])
    idx2 = jnp.concatenate([2 * src_p, 2 * src_p + 1]).reshape(-1, K)
    dst_p = dst_p.reshape(-1, K)

    xp = jnp.pad(x, ((0, 0), (0, 8 - x.shape[1])))
    Wb0p = jnp.pad(Wb0, ((0, 8 - Wb0.shape[0]), (0, 0)))
    bcol = batch.reshape(N, 1)
    bl2 = bl.reshape(1, 1)

    y0 = _mm_in(xp, Wb0p)
    g = _sc_agg(y0, idx2, dst_p)
    h1 = _addrelu(y0, g)
    g = _sc_agg(h1, idx2, dst_p)
    h2 = _layer(h1, g, Wb1)
    g = _sc_agg(h2, idx2, dst_p)
    e = _layer(h2, g, Wb2)
    g = _sc_agg(e, idx2, dst_p)
    hp, hv = _branch(e, g, Wp0, Wv0)
    gp = _sc_agg(hp, idx2, dst_p)
    pi = _layer(hp, gp, Wp1)
    gv = _sc_agg(hv, idx2, dst_p)
    vemb = _layer(hv, gv, Wv1)

    log_pi, val = _seg_all(pi, vemb, bcol, Wl, bl2)
    return (log_pi, val[:, 0])


# speedup vs baseline: 1.3137x; 1.3137x over previous
import functools

import jax
import jax.numpy as jnp
from jax import lax
from jax.experimental import pallas as pl
from jax.experimental.pallas import tpu as pltpu
from jax.experimental.pallas import tpu_sc as plsc

N = 50000
E = 800000
G = 64
H = 64
HH = H // 2

NC = 2
NS = 16
K = 128

CHUNKS = 392
EPT = CHUNKS * K
E_PAD = EPT * NS
BC = 4
NBLK = CHUNKS // BC
ZR = 56
N_PAD = 50176
NPT = N_PAD // NS
WPT = 3136

R = 10000
NB = N // R
SR = 2000
NSR = N // SR

_mesh = plsc.VectorSubcoreMesh(core_axis_name="c", subcore_axis_name="s")


@functools.partial(
    pl.kernel,
    out_type=jax.ShapeDtypeStruct((2 * N, HH), jnp.float32),
    mesh=_mesh,
    scratch_types=[
        pltpu.VMEM_SHARED((N_PAD, HH), jnp.float32),
        pltpu.VMEM((ZR, HH), jnp.float32),
        pltpu.VMEM((BC, K), jnp.int32),
        pltpu.VMEM((BC, K), jnp.int32),
        pltpu.VMEM((BC, K), jnp.int32),
        pltpu.VMEM((BC, K), jnp.int32),
        pltpu.VMEM((BC, K, HH), jnp.float32),
        pltpu.SemaphoreType.DMA((BC,)),
        pltpu.SemaphoreType.DMA((BC,)),
    ],
    compiler_params=pltpu.CompilerParams(use_tc_tiling_on_sc=False),
)
def _agg_sc(h2, idx2, dstp, out, acc, zbuf, isA, idA, isB, idB, rows, sem_g,
            sem_s):
    c = lax.axis_index("c")
    s = lax.axis_index("s")

    ib = c * (E_PAD // K) + s * CHUNKS
    db = s * CHUNKS

    def load_idx(blkno, i_buf, d_buf):
        pltpu.sync_copy(idx2.at[pl.ds(pl.multiple_of(ib + blkno * BC, 2),
                                      BC)], i_buf)
        pltpu.sync_copy(dstp.at[pl.ds(pl.multiple_of(db + blkno * BC, 2),
                                      BC)], d_buf)

    load_idx(0, isA, idA)
    for j in range(BC):
        pltpu.async_copy(h2.at[isA.at[j]], rows.at[j], sem_g.at[j])

    zero = jnp.zeros((16,), jnp.float32)

    def zb(i, carry):
        zbuf[i, 0:16] = zero
        zbuf[i, 16:32] = zero
        return carry

    lax.fori_loop(0, ZR, zb, 0)

    def za(k, carry):
        zoff = pl.multiple_of(s * NPT + k * ZR, 8)
        pltpu.sync_copy(zbuf, acc.at[pl.ds(zoff, ZR)])
        return carry

    lax.fori_loop(0, NPT // ZR, za, 0)
    plsc.subcore_barrier()

    def step(t, carry):
        load_idx(2 * t + 1, isB, idB)
        for j in range(BC):
            pltpu.make_async_copy(h2.at[isA.at[j]], rows.at[j],
                                  sem_g.at[j]).wait()
            pltpu.async_copy(rows.at[j], acc.at[idA.at[j]], sem_s.at[j],
                             add=True)
        for j in range(BC):
            pltpu.make_async_copy(rows.at[j], acc.at[idA.at[j]],
                                  sem_s.at[j]).wait()
            pltpu.async_copy(h2.at[isB.at[j]], rows.at[j], sem_g.at[j])

        @pl.when(t < NBLK // 2 - 1)
        def _():
            load_idx(2 * t + 2, isA, idA)
            for j in range(BC):
                pltpu.make_async_copy(h2.at[isB.at[j]], rows.at[j],
                                      sem_g.at[j]).wait()
                pltpu.async_copy(rows.at[j], acc.at[idB.at[j]], sem_s.at[j],
                                 add=True)
            for j in range(BC):
                pltpu.make_async_copy(rows.at[j], acc.at[idB.at[j]],
                                      sem_s.at[j]).wait()
                pltpu.async_copy(h2.at[isA.at[j]], rows.at[j], sem_g.at[j])

        @pl.when(t == NBLK // 2 - 1)
        def _():
            for j in range(BC):
                pltpu.make_async_copy(h2.at[isB.at[j]], rows.at[j],
                                      sem_g.at[j]).wait()
                pltpu.async_copy(rows.at[j], acc.at[idB.at[j]], sem_s.at[j],
                                 add=True)
            for j in range(BC):
                pltpu.make_async_copy(rows.at[j], acc.at[idB.at[j]],
                                      sem_s.at[j]).wait()

        return carry

    lax.fori_loop(0, NBLK // 2, step, 0)
    plsc.subcore_barrier()

    wbase = pl.multiple_of(jnp.minimum(s * WPT, N - WPT), 8)
    pltpu.sync_copy(acc.at[pl.ds(wbase, WPT)],
                    out.at[pl.ds(pl.multiple_of(c * N + wbase, 8), WPT)])


def _sc_agg(h, idx2, dstp):
    return _agg_sc(h.reshape(2 * N, HH), idx2, dstp)


def _mm_in(xp, Wp):
    def body(x_ref, w_ref, o_ref):
        o_ref[...] = jnp.dot(x_ref[...], w_ref[...],
                             preferred_element_type=jnp.float32)

    return pl.pallas_call(
        body,
        grid=(NB,),
        in_specs=[pl.BlockSpec((R, 8), lambda i: (i, 0)),
                  pl.BlockSpec((8, H), lambda i: (0, 0))],
        out_specs=pl.BlockSpec((R, H), lambda i: (i, 0)),
        out_shape=jax.ShapeDtypeStruct((N, H), jnp.float32),
    )(xp, Wp)


def _addrelu(y, g2):
    def body(y_ref, g0_ref, g1_ref, o_ref):
        g = jnp.concatenate([g0_ref[...], g1_ref[...]], axis=1)
        o_ref[...] = jnp.maximum(y_ref[...] + g, 0.0)

    return pl.pallas_call(
        body,
        grid=(NB,),
        in_specs=[pl.BlockSpec((R, H), lambda i: (i, 0)),
                  pl.BlockSpec((R, HH), lambda i: (i, 0)),
                  pl.BlockSpec((R, HH), lambda i: (i + NB, 0))],
        out_specs=pl.BlockSpec((R, H), lambda i: (i, 0)),
        out_shape=jax.ShapeDtypeStruct((N, H), jnp.float32),
    )(y, g2, g2)


def _layer(h, g2, W):
    def body(h_ref, g0_ref, g1_ref, w_ref, o_ref):
        t0 = h_ref[:, 0:HH] + g0_ref[...]
        t1 = h_ref[:, HH:H] + g1_ref[...]
        acc = jnp.dot(t0, w_ref[0:HH, :], preferred_element_type=jnp.float32)
        acc += jnp.dot(t1, w_ref[HH:H, :], preferred_element_type=jnp.float32)
        o_ref[...] = jnp.maximum(acc, 0.0)

    return pl.pallas_call(
        body,
        grid=(NB,),
        in_specs=[pl.BlockSpec((R, H), lambda i: (i, 0)),
                  pl.BlockSpec((R, HH), lambda i: (i, 0)),
                  pl.BlockSpec((R, HH), lambda i: (i + NB, 0)),
                  pl.BlockSpec((H, H), lambda i: (0, 0))],
        out_specs=pl.BlockSpec((R, H), lambda i: (i, 0)),
        out_shape=jax.ShapeDtypeStruct((N, H), jnp.float32),
    )(h, g2, g2, W)


def _branch(e, g2, Wp0, Wv0):
    def body(h_ref, g0_ref, g1_ref, wp_ref, wv_ref, op_ref, ov_ref):
        t0 = h_ref[:, 0:HH] + g0_ref[...]
        t1 = h_ref[:, HH:H] + g1_ref[...]
        ap = jnp.dot(t0, wp_ref[0:HH, :], preferred_element_type=jnp.float32)
        ap += jnp.dot(t1, wp_ref[HH:H, :], preferred_element_type=jnp.float32)
        op_ref[...] = jnp.maximum(ap, 0.0)
        av = jnp.dot(t0, wv_ref[0:HH, :], preferred_element_type=jnp.float32)
        av += jnp.dot(t1, wv_ref[HH:H, :], preferred_element_type=jnp.float32)
        ov_ref[...] = jnp.maximum(av, 0.0)

    return pl.pallas_call(
        body,
        grid=(NB,),
        in_specs=[pl.BlockSpec((R, H), lambda i: (i, 0)),
                  pl.BlockSpec((R, HH), lambda i: (i, 0)),
                  pl.BlockSpec((R, HH), lambda i: (i + NB, 0)),
                  pl.BlockSpec((H, H), lambda i: (0, 0)),
                  pl.BlockSpec((H, H), lambda i: (0, 0))],
        out_specs=[pl.BlockSpec((R, H), lambda i: (i, 0)),
                   pl.BlockSpec((R, H), lambda i: (i, 0))],
        out_shape=[jax.ShapeDtypeStruct((N, H), jnp.float32),
                   jax.ShapeDtypeStruct((N, H), jnp.float32)],
    )(e, g2, g2, Wp0, Wv0)


def _seg_all(pi, vemb, bcol, Wl, bl2):
    def body(pi_ref, v_ref, b_ref, wl_ref, bl_ref, lp_ref, val_ref,
             m_acc, s_acc, d_acc):
        i = pl.program_id(0)
        giota = lax.broadcasted_iota(jnp.int32, (G, 1), 0)
        giota_row = lax.broadcasted_iota(jnp.int32, (1, G), 1)
        b = b_ref[...]
        onehot = (b == giota_row).astype(jnp.float32)

        @pl.when(i == 0)
        def _():
            m_acc[...] = jnp.full((G, H), -3e38, jnp.float32)
            s_acc[...] = jnp.zeros((G, H), jnp.float32)
            d_acc[...] = jnp.zeros((G, H), jnp.float32)

        @pl.when(i < NSR)
        def _():
            s_acc[...] += lax.dot_general(
                onehot, v_ref[...], (((0,), (0,)), ((), ())),
                preferred_element_type=jnp.float32)
            pi_blk = pi_ref[...]
            g_lo = b_ref[0, 0]
            g_hi = b_ref[SR - 1, 0]

            def gbody(g, carry):
                mask = b == g
                vals = jnp.where(mask, pi_blk, -3e38)
                row = jnp.max(vals, axis=0, keepdims=True)
                sel = giota == g
                m_acc[...] = jnp.where(
                    sel,
                    jnp.maximum(m_acc[...], jnp.broadcast_to(row, (G, H))),
                    m_acc[...])
                return carry

            lax.fori_loop(g_lo, g_hi + 1, gbody, 0)

        @pl.when((i >= NSR) & (i < 2 * NSR))
        def _():
            msel = jnp.dot(onehot, m_acc[...],
                           preferred_element_type=jnp.float32)
            e = jnp.exp(pi_ref[...] - msel)
            d_acc[...] += lax.dot_general(
                onehot, e, (((0,), (0,)), ((), ())),
                preferred_element_type=jnp.float32)

        @pl.when(i >= 2 * NSR)
        def _():
            shift = m_acc[...] + jnp.log(jnp.maximum(d_acc[...], 1e-30))
            ssel = jnp.dot(onehot, shift, preferred_element_type=jnp.float32)
            lp_ref[...] = pi_ref[...] - ssel

        @pl.when(i == 3 * NSR - 1)
        def _():
            v = jnp.dot(s_acc[...], wl_ref[...],
                        preferred_element_type=jnp.float32) + bl_ref[0, 0]
            val_ref[...] = 1.0 / (1.0 + jnp.exp(-v))

    chunk = lambda i: (lax.rem(i, NSR), 0)
    return pl.pallas_call(
        body,
        grid=(3 * NSR,),
        in_specs=[pl.BlockSpec((SR, H), chunk),
                  pl.BlockSpec((SR, H), chunk),
                  pl.BlockSpec((SR, 1), chunk),
                  pl.BlockSpec((H, 1), lambda i: (0, 0)),
                  pl.BlockSpec((1, 1), lambda i: (0, 0))],
        out_specs=[pl.BlockSpec((SR, H),
                                lambda i: (jnp.where(i >= 2 * NSR,
                                                     i - 2 * NSR, 0), 0)),
                   pl.BlockSpec((G, 1), lambda i: (0, 0))],
        out_shape=[jax.ShapeDtypeStruct((N, H), jnp.float32),
                   jax.ShapeDtypeStruct((G, 1), jnp.float32)],
        scratch_shapes=[pltpu.VMEM((G, H), jnp.float32),
                        pltpu.VMEM((G, H), jnp.float32),
                        pltpu.VMEM((G, H), jnp.float32)],
    )(pi, vemb, bcol, Wl, bl2)


def kernel(x, edge_index, batch, Wb0, Wb1, Wb2, Wp0, Wp1, Wv0, Wv1, Wl, bl):
    src = edge_index[0]
    dst = edge_index[1]
    pad = E_PAD - E
    src_p = jnp.concatenate([src, jnp.zeros((pad,), jnp.int32)])
    dst_p = jnp.concatenate([dst, jnp.full((pad,), N, jnp.int32)])
    idx2 = jnp.concatenate([2 * src_p, 2 * src_p + 1]).reshape(-1, K)
    dst_p = dst_p.reshape(-1, K)

    xp = jnp.pad(x, ((0, 0), (0, 8 - x.shape[1])))
    Wb0p = jnp.pad(Wb0, ((0, 8 - Wb0.shape[0]), (0, 0)))
    bcol = batch.reshape(N, 1)
    bl2 = bl.reshape(1, 1)

    y0 = _mm_in(xp, Wb0p)
    g = _sc_agg(y0, idx2, dst_p)
    h1 = _addrelu(y0, g)
    g = _sc_agg(h1, idx2, dst_p)
    h2 = _layer(h1, g, Wb1)
    g = _sc_agg(h2, idx2, dst_p)
    e = _layer(h2, g, Wb2)
    g = _sc_agg(e, idx2, dst_p)
    hp, hv = _branch(e, g, Wp0, Wv0)
    gp = _sc_agg(hp, idx2, dst_p)
    pi = _layer(hp, gp, Wp1)
    gv = _sc_agg(hv, idx2, dst_p)
    vemb = _layer(hv, gv, Wv1)

    log_pi, val = _seg_all(pi, vemb, bcol, Wl, bl2)
    return (log_pi, val[:, 0])
